# baseline (speedup 1.0000x reference)
import functools

import jax
import jax.numpy as jnp
from jax import lax
from jax.experimental import pallas as pl
from jax.experimental.pallas import tpu as pltpu
from jax.experimental.pallas import tpu_sc as plsc

VOCAB = 1000000
DIM = 32
BATCH = 16384
HIST = 50

NC = 2
NS = 16
NW = NC * NS
RPW = BATCH // NW
CB = 16
NCHUNK = RPW // CB

_mesh = plsc.VectorSubcoreMesh(core_axis_name="c", subcore_axis_name="s")


@functools.partial(
    pl.kernel,
    mesh=_mesh,
    out_type=jax.ShapeDtypeStruct((BATCH, HIST, DIM), jnp.float32),
    scratch_types=[
        pltpu.VMEM((2, CB, HIST), jnp.int32),
        pltpu.VMEM((2, CB, HIST, DIM), jnp.float32),
        pltpu.SemaphoreType.DMA,
        pltpu.SemaphoreType.DMA,
        pltpu.SemaphoreType.DMA,
        pltpu.SemaphoreType.DMA,
        pltpu.SemaphoreType.DMA,
        pltpu.SemaphoreType.DMA,
    ],
    compiler_params=pltpu.CompilerParams(use_tc_tiling_on_sc=False),
)
def _emb_gather(
    data_hbm, table_hbm, out_hbm, idx_v, rows_v, si0, si1, sg0, sg1, so0, so1
):
    wid = lax.axis_index("s") * NC + lax.axis_index("c")
    row0 = wid * RPW
    si = (si0, si1)
    sg = (sg0, sg1)
    so = (so0, so1)

    def stage_idx(c, p):
        pltpu.async_copy(
            data_hbm.at[pl.ds(row0 + c * CB, CB)], idx_v.at[p], si[p]
        )

    def gathers(c, p):
        del c
        for i in range(CB):
            pltpu.async_copy(
                table_hbm.at[idx_v.at[p, i]], rows_v.at[p, i], sg[p]
            )

    def put(c, p):
        pltpu.async_copy(
            rows_v.at[p], out_hbm.at[pl.ds(row0 + c * CB, CB)], so[p]
        )

    def wait_idx(p):
        pltpu.make_async_copy(
            data_hbm.at[pl.ds(0, CB)], idx_v.at[p], si[p]
        ).wait()

    def wait_gathers(p):
        for i in range(CB):
            pltpu.make_async_copy(
                table_hbm.at[idx_v.at[p, i]], rows_v.at[p, i], sg[p]
            ).wait()

    def wait_put(p):
        pltpu.make_async_copy(
            rows_v.at[p], out_hbm.at[pl.ds(0, CB)], so[p]
        ).wait()

    stage_idx(0, 0)
    wait_idx(0)
    gathers(0, 0)
    stage_idx(1, 1)
    for g in range(NCHUNK):
        p = g % 2
        q = (g + 1) % 2
        wait_gathers(p)
        if g + 1 < NCHUNK:
            wait_idx(q)
            if g >= 1:
                wait_put(q)
            gathers(g + 1, q)
            if g + 2 < NCHUNK:
                stage_idx(g + 2, p)
        put(g, p)
    wait_put(0)
    wait_put(1)


def kernel(data, weight):
    return _emb_gather(data, weight)


# speedup vs baseline: 1.0076x; 1.0076x over previous
import functools

import jax
import jax.numpy as jnp
from jax import lax
from jax.experimental import pallas as pl
from jax.experimental.pallas import tpu as pltpu
from jax.experimental.pallas import tpu_sc as plsc

VOCAB = 1000000
DIM = 32
BATCH = 16384
HIST = 50

NC = 2
NS = 16
NW = NC * NS
RPW = BATCH // NW
CB = 32
NCHUNK = RPW // CB

_mesh = plsc.VectorSubcoreMesh(core_axis_name="c", subcore_axis_name="s")


@functools.partial(
    pl.kernel,
    mesh=_mesh,
    out_type=jax.ShapeDtypeStruct((BATCH, HIST, DIM), jnp.float32),
    scratch_types=[
        pltpu.VMEM((2, CB, HIST), jnp.int32),
        pltpu.VMEM((2, CB, HIST, DIM), jnp.float32),
        pltpu.SemaphoreType.DMA,
        pltpu.SemaphoreType.DMA,
        pltpu.SemaphoreType.DMA,
        pltpu.SemaphoreType.DMA,
        pltpu.SemaphoreType.DMA,
        pltpu.SemaphoreType.DMA,
    ],
    compiler_params=pltpu.CompilerParams(use_tc_tiling_on_sc=False),
)
def _emb_gather(
    data_hbm, table_hbm, out_hbm, idx_v, rows_v, si0, si1, sg0, sg1, so0, so1
):
    wid = lax.axis_index("s") * NC + lax.axis_index("c")
    row0 = wid * RPW
    si = (si0, si1)
    sg = (sg0, sg1)
    so = (so0, so1)

    def stage_idx(c, p):
        pltpu.async_copy(
            data_hbm.at[pl.ds(row0 + c * CB, CB)], idx_v.at[p], si[p]
        )

    def gathers(c, p):
        del c
        for i in range(CB):
            pltpu.async_copy(
                table_hbm.at[idx_v.at[p, i]], rows_v.at[p, i], sg[p]
            )

    def put(c, p):
        pltpu.async_copy(
            rows_v.at[p], out_hbm.at[pl.ds(row0 + c * CB, CB)], so[p]
        )

    def wait_idx(p):
        pltpu.make_async_copy(
            data_hbm.at[pl.ds(0, CB)], idx_v.at[p], si[p]
        ).wait()

    def wait_gathers(p):
        for i in range(CB):
            pltpu.make_async_copy(
                table_hbm.at[idx_v.at[p, i]], rows_v.at[p, i], sg[p]
            ).wait()

    def wait_put(p):
        pltpu.make_async_copy(
            rows_v.at[p], out_hbm.at[pl.ds(0, CB)], so[p]
        ).wait()

    stage_idx(0, 0)
    wait_idx(0)
    gathers(0, 0)
    stage_idx(1, 1)
    for g in range(NCHUNK):
        p = g % 2
        q = (g + 1) % 2
        wait_gathers(p)
        if g + 1 < NCHUNK:
            wait_idx(q)
            if g >= 1:
                wait_put(q)
            gathers(g + 1, q)
            if g + 2 < NCHUNK:
                stage_idx(g + 2, p)
        put(g, p)
    wait_put(0)
    wait_put(1)


def kernel(data, weight):
    return _emb_gather(data, weight)
